# Initial kernel scaffold; baseline (speedup 1.0000x reference)
import jax
import jax.numpy as jnp
from jax.experimental import pallas as pl


def kernel(x, edge_index, W0, b0, W1, b1, W2, b2, W3, b3, Wf1, bf1, Wf2, bf2):
    raise NotImplementedError("write your pallas kernel here")


# speedup vs baseline: 13.6908x; 13.6908x over previous
import functools

import jax
import jax.numpy as jnp
from jax import lax
from jax.experimental import pallas as pl
from jax.experimental.pallas import tpu as pltpu
from jax.experimental.pallas import tpu_sc as plsc

N = 10000
E = 320000
D = 128

NC = 2
NS = 16
NW = NC * NS

CH = 128
NCHUNK = E // CH
NP = 10112
RPT = NP // NS
_ZCHUNKS = (128, 128, 128, 128, 120)

@functools.lru_cache(maxsize=None)
def _sc_mesh():
    return plsc.VectorSubcoreMesh(
        core_axis_name="c", subcore_axis_name="s", num_cores=NC,
        num_subcores=NS,
    )


def _fill_f32(ref, nrows, ncols16, value):
    vec = jnp.full((16,), value, jnp.float32)

    def body(i, carry):
        for k in range(ncols16):
            ref[i, pl.ds(16 * k, 16)] = vec
        return carry

    lax.fori_loop(0, nrows, body, 0)


def _chunk_range(wid):
    lo = (wid * NCHUNK) // NW
    hi = ((wid + 1) * NCHUNK) // NW
    return lo, hi


@functools.lru_cache(maxsize=None)
def _sc_degree_kernel():
    return pl.kernel(
        _sc_degree_body,
        out_type=jax.ShapeDtypeStruct((NC, NP, D), jnp.float32),
        mesh=_sc_mesh(),
        scratch_types=[
            pltpu.VMEM((1, CH), jnp.int32),
            pltpu.VMEM((CH, D), jnp.float32),
            pltpu.VMEM_SHARED((NP, D), jnp.float32),
        ],
    )


def _sc_degree_body(ei, out, dst_v, buf_v, acc_sh):
    c = lax.axis_index("c")
    s = lax.axis_index("s")
    wid = s * NC + c

    _fill_f32(buf_v, CH, D // 16, 0.0)
    off = 0
    for z in _ZCHUNKS:
        pltpu.sync_copy(
            buf_v.at[pl.ds(0, z)],
            acc_sh.at[pl.ds(s * RPT + off, z)],
        )
        off += z
    plsc.subcore_barrier()

    _fill_f32(buf_v, CH, D // 16, 1.0)
    lo, hi = _chunk_range(wid)

    def body(j, carry):
        e0 = j * CH
        pltpu.sync_copy(ei.at[1, pl.ds(e0, CH)], dst_v.at[0])
        pltpu.sync_copy(buf_v, acc_sh.at[dst_v.at[0]], add=True)
        return carry

    lax.fori_loop(lo, hi, body, 0)
    plsc.subcore_barrier()

    pltpu.sync_copy(
        acc_sh.at[pl.ds(s * RPT, RPT)],
        out.at[c, pl.ds(s * RPT, RPT)],
    )


@functools.lru_cache(maxsize=None)
def _sc_edge_sum_kernel():
    return pl.kernel(
        _sc_edge_sum_body,
        out_type=jax.ShapeDtypeStruct((NC, NP, D), jnp.float32),
        mesh=_sc_mesh(),
        scratch_types=[
            pltpu.VMEM((CH,), jnp.int32),
            pltpu.VMEM((1, CH), jnp.int32),
            pltpu.VMEM((CH, D), jnp.float32),
            pltpu.VMEM_SHARED((NP, D), jnp.float32),
            pltpu.SemaphoreType.DMA,
        ],
    )


def _sc_edge_sum_body(g, ei, out, src_v, dst_v, rows_v, acc_sh, sem):
    c = lax.axis_index("c")
    s = lax.axis_index("s")
    wid = s * NC + c

    _fill_f32(rows_v, CH, D // 16, 0.0)
    off = 0
    for z in _ZCHUNKS:
        pltpu.sync_copy(
            rows_v.at[pl.ds(0, z)],
            acc_sh.at[pl.ds(s * RPT + off, z)],
        )
        off += z
    plsc.subcore_barrier()

    lo, hi = _chunk_range(wid)

    def body(j, carry):
        e0 = j * CH
        pltpu.sync_copy(ei.at[0, pl.ds(e0, CH)], src_v)
        pltpu.sync_copy(ei.at[1, pl.ds(e0, CH)], dst_v.at[0])
        pltpu.async_copy(g.at[src_v], rows_v, sem).wait()
        pltpu.sync_copy(rows_v, acc_sh.at[dst_v.at[0]], add=True)
        return carry

    lax.fori_loop(lo, hi, body, 0)
    plsc.subcore_barrier()

    pltpu.sync_copy(
        acc_sh.at[pl.ds(s * RPT, RPT)],
        out.at[c, pl.ds(s * RPT, RPT)],
    )


RB = 2000
GRID = N // RB


def _tc_dinv_body(dp_ref, dv_ref):
    deg = dp_ref[0][:, 0:1] + dp_ref[1][:, 0:1] + 1.0
    dv_ref[...] = lax.rsqrt(deg)


def _tc_head_body(x_ref, dv_ref, w0_ref, b0_ref, w1_ref, g1_ref):
    dinv = dv_ref[...]
    h0 = jax.nn.sigmoid(
        jnp.dot(x_ref[...], w0_ref[...], preferred_element_type=jnp.float32)
        + b0_ref[...]
    )
    g1_ref[...] = dinv * jnp.dot(
        h0, w1_ref[...], preferred_element_type=jnp.float32
    )


def _tc_mid_body(sp_ref, g_ref, dv_ref, b_ref, wn_ref, gn_ref):
    dinv = dv_ref[...]
    conv = dinv * (sp_ref[0] + sp_ref[1] + g_ref[...]) + b_ref[...]
    h = jax.nn.sigmoid(conv)
    gn_ref[...] = dinv * jnp.dot(
        h, wn_ref[...], preferred_element_type=jnp.float32
    )


def _tc_tail_body(sp_ref, g_ref, dv_ref, b_ref, wf1_ref, bf1_ref, wf2_ref,
                  bf2_ref, out_ref):
    dinv = dv_ref[...]
    conv = dinv * (sp_ref[0] + sp_ref[1] + g_ref[...]) + b_ref[...]
    h = jax.nn.sigmoid(conv)
    t = jax.nn.sigmoid(
        jnp.dot(h, wf1_ref[...], preferred_element_type=jnp.float32)
        + bf1_ref[...]
    )
    out_ref[...] = (
        jnp.dot(t, wf2_ref[...], preferred_element_type=jnp.float32)
        + bf2_ref[...]
    )


def _row_spec(cols):
    return pl.BlockSpec((RB, cols), lambda i: (i, 0))


def _part_spec(cols):
    return pl.BlockSpec((NC, RB, cols), lambda i: (0, i, 0))


def _full_spec(r, cols):
    return pl.BlockSpec((r, cols), lambda i: (0, 0))


_tc_dinv = pl.pallas_call(
    _tc_dinv_body,
    grid=(GRID,),
    in_specs=[_part_spec(D)],
    out_specs=_row_spec(1),
    out_shape=jax.ShapeDtypeStruct((N, 1), jnp.float32),
)

_tc_head = pl.pallas_call(
    _tc_head_body,
    grid=(GRID,),
    in_specs=[_row_spec(D), _row_spec(1), _full_spec(D, D),
              _full_spec(1, D), _full_spec(D, D)],
    out_specs=_row_spec(D),
    out_shape=jax.ShapeDtypeStruct((N, D), jnp.float32),
)

_tc_mid = pl.pallas_call(
    _tc_mid_body,
    grid=(GRID,),
    in_specs=[_part_spec(D), _row_spec(D), _row_spec(1),
              _full_spec(1, D), _full_spec(D, D)],
    out_specs=_row_spec(D),
    out_shape=jax.ShapeDtypeStruct((N, D), jnp.float32),
)

_tc_tail = pl.pallas_call(
    _tc_tail_body,
    grid=(GRID,),
    in_specs=[_part_spec(D), _row_spec(D), _row_spec(1),
              _full_spec(1, D), _full_spec(D, 256), _full_spec(1, 256),
              _full_spec(256, 1), _full_spec(1, 1)],
    out_specs=_row_spec(1),
    out_shape=jax.ShapeDtypeStruct((N, 1), jnp.float32),
)


def kernel(x, edge_index, W0, b0, W1, b1, W2, b2, W3, b3, Wf1, bf1, Wf2, bf2):
    b0 = b0.reshape(1, D)
    b1 = b1.reshape(1, D)
    b2 = b2.reshape(1, D)
    b3 = b3.reshape(1, D)
    bf1 = bf1.reshape(1, 256)
    bf2 = bf2.reshape(1, 1)

    sc_degree = _sc_degree_kernel()
    sc_edge_sum = _sc_edge_sum_kernel()

    deg_p = sc_degree(edge_index)
    dinv = _tc_dinv(deg_p)

    g1 = _tc_head(x, dinv, W0, b0, W1)
    s1 = sc_edge_sum(g1, edge_index)
    g2 = _tc_mid(s1, g1, dinv, b1, W2)
    s2 = sc_edge_sum(g2, edge_index)
    g3 = _tc_mid(s2, g2, dinv, b2, W3)
    s3 = sc_edge_sum(g3, edge_index)
    return _tc_tail(s3, g3, dinv, b3, Wf1, bf1, Wf2, bf2)


# speedup vs baseline: 28.8703x; 2.1087x over previous
import functools

import jax
import jax.numpy as jnp
from jax import lax
from jax.experimental import pallas as pl
from jax.experimental.pallas import tpu as pltpu
from jax.experimental.pallas import tpu_sc as plsc

N = 10000
E = 320000
D = 128

NC = 2
NS = 16
NW = NC * NS

CH = 128
NT = 80
EP = NW * NT * CH
NCHUNK = EP // CH
NP = 10240
RPT = NP // NS
_ZCHUNKS = (128, 128, 128, 128, 128)

@functools.lru_cache(maxsize=None)
def _sc_mesh():
    return plsc.VectorSubcoreMesh(
        core_axis_name="c", subcore_axis_name="s", num_cores=NC,
        num_subcores=NS,
    )


def _fill_f32(ref, nrows, ncols16, value):
    vec = jnp.full((16,), value, jnp.float32)

    def body(i, carry):
        for k in range(ncols16):
            ref[i, pl.ds(16 * k, 16)] = vec
        return carry

    lax.fori_loop(0, nrows, body, 0)


@functools.lru_cache(maxsize=None)
def _sc_degree_kernel():
    return pl.kernel(
        _sc_degree_body,
        out_type=jax.ShapeDtypeStruct((NC * NP,), jnp.float32),
        mesh=_sc_mesh(),
        scratch_types=[
            pltpu.VMEM((NT, CH), jnp.int32),
            pltpu.VMEM((CH,), jnp.float32),
            pltpu.VMEM((RPT,), jnp.float32),
            pltpu.VMEM_SHARED((NP,), jnp.float32),
        ],
    )


def _sc_degree_body(dst2d, out, dsts_v, ones_v, zero_v, acc_sh):
    c = lax.axis_index("c")
    s = lax.axis_index("s")
    wid = s * NC + c

    one = jnp.ones((16,), jnp.float32)
    zero = jnp.zeros((16,), jnp.float32)
    for k in range(CH // 16):
        ones_v[pl.ds(16 * k, 16)] = one

    def zfill(i, carry):
        zero_v[pl.ds(i * 16, 16)] = zero
        return carry

    lax.fori_loop(0, RPT // 16, zfill, 0)
    pltpu.sync_copy(zero_v, acc_sh.at[pl.ds(s * RPT, RPT)])
    plsc.subcore_barrier()

    pltpu.sync_copy(dst2d.at[pl.ds(wid * NT, NT)], dsts_v)

    def body(j, carry):
        pltpu.sync_copy(ones_v, acc_sh.at[dsts_v.at[j]], add=True)
        return carry

    lax.fori_loop(0, NT, body, 0)
    plsc.subcore_barrier()

    pltpu.sync_copy(
        acc_sh.at[pl.ds(s * RPT, RPT)],
        out.at[pl.ds(c * NP + s * RPT, RPT)],
    )


@functools.lru_cache(maxsize=None)
def _sc_edge_sum_kernel():
    return pl.kernel(
        _sc_edge_sum_body,
        out_type=jax.ShapeDtypeStruct((NC, NP, D), jnp.float32),
        mesh=_sc_mesh(),
        scratch_types=[
            pltpu.VMEM((NT // 2, CH), jnp.int32),
            pltpu.VMEM((NT // 2, CH), jnp.int32),
            pltpu.VMEM((CH, D), jnp.float32),
            pltpu.VMEM((CH, D), jnp.float32),
            pltpu.VMEM_SHARED((NP, D), jnp.float32),
            pltpu.SemaphoreType.DMA,
            pltpu.SemaphoreType.DMA,
        ],
    )


def _sc_edge_sum_body(g, src2d, dst2d, out, srcs_v, dsts_v, rows0, rows1,
                      acc_sh, sem0, sem1):
    c = lax.axis_index("c")
    s = lax.axis_index("s")
    wid = s * NC + c

    _fill_f32(rows0, CH, D // 16, 0.0)
    off = 0
    for z in _ZCHUNKS:
        pltpu.sync_copy(
            rows0.at[pl.ds(0, z)],
            acc_sh.at[pl.ds(s * RPT + off, z)],
        )
        off += z
    plsc.subcore_barrier()

    NH = NT // 2
    NLIVE = E // CH
    hi = jnp.clip(NLIVE - wid * NT, 0, NT)

    for h in range(2):
        nh_live = jnp.clip(hi - h * NH, 0, NH)

        def group(gi, carry):
            j0 = gi * 2

            pltpu.make_async_copy(g.at[pl.ds(0, CH)], rows0, sem0).wait()
            pltpu.sync_copy(rows0, acc_sh.at[dsts_v.at[j0]], add=True)

            @pl.when(j0 + 2 < nh_live)
            def _():
                pltpu.async_copy(g.at[srcs_v.at[j0 + 2]], rows0, sem0)

            pltpu.make_async_copy(g.at[pl.ds(0, CH)], rows1, sem1).wait()
            pltpu.sync_copy(rows1, acc_sh.at[dsts_v.at[j0 + 1]], add=True)

            @pl.when(j0 + 3 < nh_live)
            def _():
                pltpu.async_copy(g.at[srcs_v.at[j0 + 3]], rows1, sem1)

            return carry

        pltpu.sync_copy(src2d.at[pl.ds(wid * NT + h * NH, NH)], srcs_v)
        pltpu.sync_copy(dst2d.at[pl.ds(wid * NT + h * NH, NH)], dsts_v)

        @pl.when(nh_live > 0)
        def _():
            pltpu.async_copy(g.at[srcs_v.at[0]], rows0, sem0)

        @pl.when(nh_live > 1)
        def _():
            pltpu.async_copy(g.at[srcs_v.at[1]], rows1, sem1)

        lax.fori_loop(0, nh_live // 2, group, 0)

    plsc.subcore_barrier()

    pltpu.sync_copy(
        acc_sh.at[pl.ds(s * RPT, RPT)],
        out.at[c, pl.ds(s * RPT, RPT)],
    )


RB = 2000
GRID = N // RB


def _dinv(dp_ref):
    return lax.rsqrt(dp_ref[0] + dp_ref[1] + 1.0)


def _tc_head_body(x_ref, dp_ref, w0_ref, b0_ref, w1_ref, g1_ref):
    dinv = _dinv(dp_ref)
    h0 = jax.nn.sigmoid(
        jnp.dot(x_ref[...], w0_ref[...], preferred_element_type=jnp.float32)
        + b0_ref[...]
    )
    g1_ref[...] = dinv * jnp.dot(
        h0, w1_ref[...], preferred_element_type=jnp.float32
    )


def _tc_mid_body(sp_ref, g_ref, dp_ref, b_ref, wn_ref, gn_ref):
    dinv = _dinv(dp_ref)
    conv = dinv * (sp_ref[0] + sp_ref[1] + g_ref[...]) + b_ref[...]
    h = jax.nn.sigmoid(conv)
    gn_ref[...] = dinv * jnp.dot(
        h, wn_ref[...], preferred_element_type=jnp.float32
    )


def _tc_tail_body(sp_ref, g_ref, dp_ref, b_ref, wf1_ref, bf1_ref, wf2_ref,
                  bf2_ref, out_ref):
    dinv = _dinv(dp_ref)
    conv = dinv * (sp_ref[0] + sp_ref[1] + g_ref[...]) + b_ref[...]
    h = jax.nn.sigmoid(conv)
    t = jax.nn.sigmoid(
        jnp.dot(h, wf1_ref[...], preferred_element_type=jnp.float32)
        + bf1_ref[...]
    )
    out_ref[...] = (
        jnp.dot(t, wf2_ref[...], preferred_element_type=jnp.float32)
        + bf2_ref[...]
    )


def _row_spec(cols):
    return pl.BlockSpec((RB, cols), lambda i: (i, 0))


def _part_spec(cols):
    return pl.BlockSpec((NC, RB, cols), lambda i: (0, i, 0))


def _full_spec(r, cols):
    return pl.BlockSpec((r, cols), lambda i: (0, 0))


_tc_head = pl.pallas_call(
    _tc_head_body,
    grid=(GRID,),
    in_specs=[_row_spec(D), _part_spec(1), _full_spec(D, D),
              _full_spec(1, D), _full_spec(D, D)],
    out_specs=_row_spec(D),
    out_shape=jax.ShapeDtypeStruct((NP, D), jnp.float32),
)

_tc_mid = pl.pallas_call(
    _tc_mid_body,
    grid=(GRID,),
    in_specs=[_part_spec(D), _row_spec(D), _part_spec(1),
              _full_spec(1, D), _full_spec(D, D)],
    out_specs=_row_spec(D),
    out_shape=jax.ShapeDtypeStruct((NP, D), jnp.float32),
)

_tc_tail = pl.pallas_call(
    _tc_tail_body,
    grid=(GRID,),
    in_specs=[_part_spec(D), _row_spec(D), _part_spec(1),
              _full_spec(1, D), _full_spec(D, 256), _full_spec(1, 256),
              _full_spec(256, 1), _full_spec(1, 1)],
    out_specs=_row_spec(1),
    out_shape=jax.ShapeDtypeStruct((N, 1), jnp.float32),
)


def kernel(x, edge_index, W0, b0, W1, b1, W2, b2, W3, b3, Wf1, bf1, Wf2, bf2):
    b0 = b0.reshape(1, D)
    b1 = b1.reshape(1, D)
    b2 = b2.reshape(1, D)
    b3 = b3.reshape(1, D)
    bf1 = bf1.reshape(1, 256)
    bf2 = bf2.reshape(1, 1)

    sc_degree = _sc_degree_kernel()
    sc_edge_sum = _sc_edge_sum_kernel()

    pad_rows = N + (jnp.arange(EP - E, dtype=jnp.int32) % (NP - N))
    pad = jnp.broadcast_to(pad_rows, (2, EP - E))
    ei = jnp.concatenate([edge_index, pad], axis=1)
    src2d = ei[0].reshape(NCHUNK, CH)
    dst2d = ei[1].reshape(NCHUNK, CH)

    deg_p = sc_degree(dst2d).reshape(NC, NP, 1)

    g1 = _tc_head(x, deg_p, W0, b0, W1)
    s1 = sc_edge_sum(g1, src2d, dst2d)
    g2 = _tc_mid(s1, g1, deg_p, b1, W2)
    s2 = sc_edge_sum(g2, src2d, dst2d)
    g3 = _tc_mid(s2, g2, deg_p, b2, W3)
    s3 = sc_edge_sum(g3, src2d, dst2d)
    return _tc_tail(s3, g3, deg_p, b3, Wf1, bf1, Wf2, bf2)
